# baseline (speedup 1.0000x reference)
import functools

import jax
import jax.numpy as jnp
import numpy as np
from jax import lax
from jax.experimental import pallas as pl
from jax.experimental.pallas import tpu as pltpu
from jax.experimental.pallas import tpu_sc as plsc

ALPHA = 0.05
B, N, D = 4, 8192, 1024
K = N // 2
LANES = 128
ROWS = N // LANES
TOPROWS = K // LANES

NW = 32
SUB = D // LANES
PPW = K // NW
QPW = PPW * B * SUB
CR = 32
NSTEPS = PPW * B // CR
NBUF = 3

MR = 1024


def _topk_body(w_ref, gidx_ref, tv_ref, ti_ref):
    v = w_ref[...]
    r = lax.broadcasted_iota(jnp.int32, (B * ROWS, LANES), 0)
    c = lax.broadcasted_iota(jnp.int32, (B * ROWS, LANES), 1)
    lin = (r % ROWS) * LANES + c
    idx = lin

    size = 2
    while size <= N:
        d = size // 2
        while d >= 1:
            if d < LANES:
                ax, s = 1, d
            else:
                ax, s = 0, d // LANES
            low = (lin & d) == 0
            pv = jnp.where(low, jnp.roll(v, -s, axis=ax), jnp.roll(v, s, axis=ax))
            pi = jnp.where(low, jnp.roll(idx, -s, axis=ax), jnp.roll(idx, s, axis=ax))
            first = (v > pv) | ((v == pv) & (idx < pi))
            asc = (lin & size) != 0
            take_self = first ^ (~low) ^ asc
            v = jnp.where(take_self, v, pv)
            idx = jnp.where(take_self, idx, pi)
            d //= 2
        size *= 2

    blocks = []
    for b in range(B):
        blocks.append(idx[b * ROWS:b * ROWS + TOPROWS, :] + b * N)
        at_kth = lin[:ROWS, :] == (K - 1)
        vb = v[b * ROWS:(b + 1) * ROWS, :]
        ib = idx[b * ROWS:(b + 1) * ROWS, :]
        tv_ref[b] = jnp.sum(jnp.where(at_kth, vb, jnp.zeros_like(vb)))
        ti_ref[b] = jnp.sum(jnp.where(at_kth, ib, jnp.zeros_like(ib)))
    gidx_ref[...] = jnp.concatenate(blocks, axis=0)


def _topk_call(w2):
    return pl.pallas_call(
        _topk_body,
        out_shape=[
            jax.ShapeDtypeStruct((B * TOPROWS, LANES), jnp.int32),
            jax.ShapeDtypeStruct((B,), jnp.float32),
            jax.ShapeDtypeStruct((B,), jnp.int32),
        ],
        in_specs=[pl.BlockSpec((B * ROWS, LANES), lambda: (0, 0))],
        out_specs=[
            pl.BlockSpec((B * TOPROWS, LANES), lambda: (0, 0)),
            pl.BlockSpec(memory_space=pltpu.SMEM),
            pl.BlockSpec(memory_space=pltpu.SMEM),
        ],
    )(w2)


def _msum_body(tv_ref, ti_ref, a_ref, seq_ref, o_ref):
    b = pl.program_id(0)
    s = pl.program_id(1)
    t = tv_ref[b]
    bt = ti_ref[b]
    a = a_ref[pl.ds(b, 1), pl.ds(s * MR, MR)]
    j = lax.broadcasted_iota(jnp.int32, (1, MR), 1) + s * MR
    sel = (a > t) | ((a == t) & (j <= bt))
    m = jnp.where(sel, 0.0, 1.0).astype(jnp.bfloat16)
    part = jnp.dot(m, seq_ref[0].astype(jnp.bfloat16),
                   preferred_element_type=jnp.float32)

    @pl.when(s == 0)
    def _():
        o_ref[...] = jnp.zeros_like(o_ref)

    o_ref[...] += part[None]


def _masked_sum_call(tv, ti, attn, seq):
    return pl.pallas_call(
        _msum_body,
        out_shape=jax.ShapeDtypeStruct((B, 1, D), jnp.float32),
        grid=(B, N // MR),
        in_specs=[
            pl.BlockSpec(memory_space=pltpu.SMEM),
            pl.BlockSpec(memory_space=pltpu.SMEM),
            pl.BlockSpec((B, N), lambda b, s: (0, 0)),
            pl.BlockSpec((1, MR, D), lambda b, s: (b, s, 0)),
        ],
        out_specs=pl.BlockSpec((1, 1, D), lambda b, s: (b, 0, 0)),
        compiler_params=pltpu.CompilerParams(
            dimension_semantics=("parallel", "arbitrary")),
    )(tv, ti, attn, seq)


def _sc_gather_call(seq_flat, gidx4, dq):
    mesh = plsc.VectorSubcoreMesh(core_axis_name="c", subcore_axis_name="s")

    @functools.partial(
        pl.kernel,
        out_type=jax.ShapeDtypeStruct(((K + 1) * B * SUB, LANES), jnp.float32),
        mesh=mesh,
        scratch_types=[
            pltpu.VMEM((NSTEPS, CR), jnp.int32),
            pltpu.VMEM((NSTEPS * SUB, CR), jnp.int32),
            pltpu.VMEM((CR, D), jnp.float32),
            pltpu.VMEM((CR, D), jnp.float32),
            pltpu.VMEM((CR, D), jnp.float32),
            pltpu.SemaphoreType.DMA,
            pltpu.SemaphoreType.DMA,
            pltpu.SemaphoreType.DMA,
            pltpu.SemaphoreType.DMA,
            pltpu.SemaphoreType.DMA,
            pltpu.SemaphoreType.DMA,
        ],
    )
    def k(seq_hbm, gidx_hbm, dq_hbm, out_hbm,
          idx_v, dq_v, buf0, buf1, buf2, g0, g1, g2, w0, w1, w2):
        wid = lax.axis_index("s") * 2 + lax.axis_index("c")

        for b in range(B):
            pltpu.sync_copy(gidx_hbm.at[b, wid],
                            idx_v.at[pl.ds(b * (NSTEPS // B), NSTEPS // B)])
        pltpu.sync_copy(dq_hbm.at[wid], dq_v)

        bufs = (buf0, buf1, buf2)
        gsems = (g0, g1, g2)
        wsems = (w0, w1, w2)

        def gather(t):
            pltpu.make_async_copy(seq_hbm.at[idx_v.at[t]], bufs[t % NBUF],
                                  gsems[t % NBUF]).start()

        def scatter(t, wait):
            buf, ws = bufs[t % NBUF], wsems[t % NBUF]
            for j in range(SUB):
                cp = pltpu.make_async_copy(
                    buf.at[:, pl.ds(j * LANES, LANES)],
                    out_hbm.at[dq_v.at[t * SUB + j]], ws)
                if wait:
                    cp.wait()
                else:
                    cp.start()

        gather(0)
        gather(1)
        for t in range(NSTEPS):
            pltpu.make_async_copy(seq_hbm.at[idx_v.at[t]], bufs[t % NBUF],
                                  gsems[t % NBUF]).wait()
            scatter(t, wait=False)
            if t + 2 < NSTEPS:
                if t - 1 >= 0:
                    scatter(t - 1, wait=True)
                gather(t + 2)
        for t in range(NSTEPS - 3, NSTEPS):
            scatter(t, wait=True)

    return k(seq_flat, gidx4, dq)


def _fin_body(orow_ref, rsum_ref, o_ref):
    cnt = jnp.float32(K) + jnp.float32(1e-10)
    row = jnp.float32(ALPHA) * (rsum_ref[...] / cnt)
    parts = []
    for q in range(B * SUB):
        j, b = q // B, q % B
        parts.append(row[b:b + 1, j * LANES:(j + 1) * LANES])
    o_ref[...] = jnp.concatenate(parts, axis=0)


def _finalize_call(out1, rsum):
    return pl.pallas_call(
        _fin_body,
        out_shape=jax.ShapeDtypeStruct(((K + 1) * B * SUB, LANES), jnp.float32),
        grid=(1,),
        in_specs=[
            pl.BlockSpec((B * SUB, LANES), lambda i: (K, 0)),
            pl.BlockSpec((B, D), lambda i: (0, 0)),
        ],
        out_specs=pl.BlockSpec((B * SUB, LANES), lambda i: (K, 0)),
        input_output_aliases={0: 0},
    )(out1, rsum)


def _dq_table():
    w = np.arange(NW)[:, None, None, None]
    c = np.arange(NSTEPS)[None, :, None, None]
    j = np.arange(SUB)[None, None, :, None]
    i = np.arange(CR)[None, None, None, :]
    b, rb = c // (NSTEPS // B), c % (NSTEPS // B)
    q = (w * PPW + rb * CR + i) * (B * SUB) + j * B + b
    return jnp.asarray(q.reshape(NW, NSTEPS * SUB, CR), dtype=jnp.int32)


def kernel(seq, attn_weights):
    if attn_weights.ndim == 3:
        attn_weights = jnp.squeeze(attn_weights, axis=1)
    w2 = attn_weights.reshape(B * ROWS, LANES)
    gidx, tv, ti = _topk_call(w2)
    rsum = _masked_sum_call(tv, ti, attn_weights, seq)
    out1 = _sc_gather_call(
        seq.reshape(B * N, D),
        gidx.reshape(B, NW, NSTEPS // B, CR),
        _dq_table())
    out_flat = _finalize_call(out1, rsum.reshape(B, D))
    return (out_flat.reshape(K + 1, SUB, B, LANES)
            .transpose(2, 0, 1, 3)
            .reshape(B, K + 1, D))


# speedup vs baseline: 1.0417x; 1.0417x over previous
import functools

import jax
import jax.numpy as jnp
import numpy as np
from jax import lax
from jax.experimental import pallas as pl
from jax.experimental.pallas import tpu as pltpu
from jax.experimental.pallas import tpu_sc as plsc

ALPHA = 0.05
B, N, D = 4, 8192, 1024
K = N // 2
LANES = 128
ROWS = N // LANES
TOPROWS = K // LANES

NW = 32
SUB = D // LANES
PPW = K // NW
QPW = PPW * B * SUB
CR = 32
NSTEPS = PPW * B // CR
NBUF = 3

MR = 1024


def _topk_body(w_ref, gidx_ref, tv_ref, ti_ref):
    v = w_ref[...]
    r = lax.broadcasted_iota(jnp.int32, (B * ROWS, LANES), 0)
    c = lax.broadcasted_iota(jnp.int32, (B * ROWS, LANES), 1)
    lin = (r % ROWS) * LANES + c
    idx = lin

    size = 2
    while size <= N:
        d = size // 2
        while d >= 1:
            if d < LANES:
                ax, s = 1, d
            else:
                ax, s = 0, d // LANES
            low = (lin & d) == 0
            pv = jnp.where(low, jnp.roll(v, -s, axis=ax), jnp.roll(v, s, axis=ax))
            pi = jnp.where(low, jnp.roll(idx, -s, axis=ax), jnp.roll(idx, s, axis=ax))
            first = (v > pv) | ((v == pv) & (idx < pi))
            asc = (lin & size) != 0
            take_self = first ^ (~low) ^ asc
            v = jnp.where(take_self, v, pv)
            idx = jnp.where(take_self, idx, pi)
            d //= 2
        size *= 2

    blocks = []
    for b in range(B):
        blocks.append(idx[b * ROWS:b * ROWS + TOPROWS, :] + b * N)
        at_kth = lin[:ROWS, :] == (K - 1)
        vb = v[b * ROWS:(b + 1) * ROWS, :]
        ib = idx[b * ROWS:(b + 1) * ROWS, :]
        tv_ref[b] = jnp.sum(jnp.where(at_kth, vb, jnp.zeros_like(vb)))
        ti_ref[b] = jnp.sum(jnp.where(at_kth, ib, jnp.zeros_like(ib)))
    gidx_ref[...] = jnp.concatenate(blocks, axis=0)


def _topk_call(w2):
    return pl.pallas_call(
        _topk_body,
        out_shape=[
            jax.ShapeDtypeStruct((B * TOPROWS, LANES), jnp.int32),
            jax.ShapeDtypeStruct((B,), jnp.float32),
            jax.ShapeDtypeStruct((B,), jnp.int32),
        ],
        in_specs=[pl.BlockSpec((B * ROWS, LANES), lambda: (0, 0))],
        out_specs=[
            pl.BlockSpec((B * TOPROWS, LANES), lambda: (0, 0)),
            pl.BlockSpec(memory_space=pltpu.SMEM),
            pl.BlockSpec(memory_space=pltpu.SMEM),
        ],
    )(w2)


def _msum_body(tv_ref, ti_ref, a_ref, seq_ref, o_ref):
    b = pl.program_id(0)
    s = pl.program_id(1)
    t = tv_ref[b]
    bt = ti_ref[b]
    a = a_ref[pl.ds(b, 1), pl.ds(s * MR, MR)]
    j = lax.broadcasted_iota(jnp.int32, (1, MR), 1) + s * MR
    sel = (a > t) | ((a == t) & (j <= bt))
    m = jnp.where(sel, 0.0, 1.0).astype(jnp.bfloat16)
    part = jnp.dot(m, seq_ref[0].astype(jnp.bfloat16),
                   preferred_element_type=jnp.float32)

    @pl.when(s == 0)
    def _():
        o_ref[...] = jnp.zeros_like(o_ref)

    o_ref[...] += part[None]


def _masked_sum_call(tv, ti, attn, seq):
    return pl.pallas_call(
        _msum_body,
        out_shape=jax.ShapeDtypeStruct((B, 1, D), jnp.float32),
        grid=(B, N // MR),
        in_specs=[
            pl.BlockSpec(memory_space=pltpu.SMEM),
            pl.BlockSpec(memory_space=pltpu.SMEM),
            pl.BlockSpec((B, N), lambda b, s: (0, 0)),
            pl.BlockSpec((1, MR, D), lambda b, s: (b, s, 0)),
        ],
        out_specs=pl.BlockSpec((1, 1, D), lambda b, s: (b, 0, 0)),
        compiler_params=pltpu.CompilerParams(
            dimension_semantics=("parallel", "arbitrary")),
    )(tv, ti, attn, seq)


def _sc_gather_call(seq_flat, gidx2):
    mesh = plsc.VectorSubcoreMesh(core_axis_name="c", subcore_axis_name="s")
    L = 16

    @functools.partial(
        pl.kernel,
        out_type=jax.ShapeDtypeStruct(((K + 1) * B * SUB, LANES), jnp.float32),
        mesh=mesh,
        scratch_types=[
            pltpu.VMEM((B, PPW), jnp.int32),
            pltpu.VMEM((NSTEPS * SUB, CR), jnp.int32),
            pltpu.VMEM((CR, D), jnp.float32),
            pltpu.VMEM((CR, D), jnp.float32),
            pltpu.VMEM((CR, D), jnp.float32),
            pltpu.SemaphoreType.DMA,
            pltpu.SemaphoreType.DMA,
            pltpu.SemaphoreType.DMA,
            pltpu.SemaphoreType.DMA,
            pltpu.SemaphoreType.DMA,
            pltpu.SemaphoreType.DMA,
        ],
    )
    def k(seq_hbm, gidx_hbm, out_hbm,
          idx_v, dq_v, buf0, buf1, buf2, g0, g1, g2, w0, w1, w2):
        wid = lax.axis_index("s") * 2 + lax.axis_index("c")

        for b in range(B):
            pltpu.sync_copy(gidx_hbm.at[b * NW + wid], idx_v.at[b])

        iota = lax.broadcasted_iota(jnp.int32, (L,), 0) * (B * SUB)
        for c in range(NSTEPS):
            b, rb = c // (NSTEPS // B), c % (NSTEPS // B)
            base = (wid * PPW + rb * CR) * (B * SUB) + b
            for j in range(SUB):
                for h in range(CR // L):
                    dq_v[c * SUB + j, pl.ds(h * L, L)] = (
                        iota + (base + h * L * B * SUB + j * B))

        bufs = (buf0, buf1, buf2)
        gsems = (g0, g1, g2)
        wsems = (w0, w1, w2)

        def gather(t):
            b, rb = t // (NSTEPS // B), t % (NSTEPS // B)
            pltpu.make_async_copy(
                seq_hbm.at[idx_v.at[b, pl.ds(rb * CR, CR)]],
                bufs[t % NBUF], gsems[t % NBUF]).start()

        def gather_wait(t):
            b, rb = t // (NSTEPS // B), t % (NSTEPS // B)
            pltpu.make_async_copy(
                seq_hbm.at[idx_v.at[b, pl.ds(rb * CR, CR)]],
                bufs[t % NBUF], gsems[t % NBUF]).wait()

        def scatter(t, wait):
            buf, ws = bufs[t % NBUF], wsems[t % NBUF]
            for j in range(SUB):
                cp = pltpu.make_async_copy(
                    buf.at[:, pl.ds(j * LANES, LANES)],
                    out_hbm.at[dq_v.at[t * SUB + j]], ws)
                if wait:
                    cp.wait()
                else:
                    cp.start()

        gather(0)
        gather(1)
        for t in range(NSTEPS):
            gather_wait(t)
            scatter(t, wait=False)
            if t + 2 < NSTEPS:
                if t - 1 >= 0:
                    scatter(t - 1, wait=True)
                gather(t + 2)
        for t in range(NSTEPS - 3, NSTEPS):
            scatter(t, wait=True)

    return k(seq_flat, gidx2)


def _fin_body(orow_ref, rsum_ref, o_ref):
    cnt = jnp.float32(K) + jnp.float32(1e-10)
    row = jnp.float32(ALPHA) * (rsum_ref[...] / cnt)
    parts = []
    for q in range(B * SUB):
        j, b = q // B, q % B
        parts.append(row[b:b + 1, j * LANES:(j + 1) * LANES])
    o_ref[...] = jnp.concatenate(parts, axis=0)


def _finalize_call(out1, rsum):
    return pl.pallas_call(
        _fin_body,
        out_shape=jax.ShapeDtypeStruct(((K + 1) * B * SUB, LANES), jnp.float32),
        grid=(1,),
        in_specs=[
            pl.BlockSpec((B * SUB, LANES), lambda i: (K, 0)),
            pl.BlockSpec((B, D), lambda i: (0, 0)),
        ],
        out_specs=pl.BlockSpec((B * SUB, LANES), lambda i: (K, 0)),
        input_output_aliases={0: 0},
    )(out1, rsum)


def kernel(seq, attn_weights):
    if attn_weights.ndim == 3:
        attn_weights = jnp.squeeze(attn_weights, axis=1)
    w2 = attn_weights.reshape(B * ROWS, LANES)
    gidx, tv, ti = _topk_call(w2)
    rsum = _masked_sum_call(tv, ti, attn_weights, seq)
    out1 = _sc_gather_call(seq.reshape(B * N, D), gidx)
    out_flat = _finalize_call(out1, rsum.reshape(B, D))
    return (out_flat.reshape(K + 1, SUB, B, LANES)
            .transpose(2, 0, 1, 3)
            .reshape(B, K + 1, D))


# speedup vs baseline: 1.0601x; 1.0176x over previous
import functools

import jax
import jax.numpy as jnp
import numpy as np
from jax import lax
from jax.experimental import pallas as pl
from jax.experimental.pallas import tpu as pltpu
from jax.experimental.pallas import tpu_sc as plsc

ALPHA = 0.05
B, N, D = 4, 8192, 1024
K = N // 2
LANES = 128
ROWS = N // LANES
TOPROWS = K // LANES

NW = 32
SUB = D // LANES
PPW = K // NW
QPW = PPW * B * SUB
CR = 32
NSTEPS = PPW * B // CR
NBUF = 3

MR = 2048


def _topk_body(w_ref, gidx_ref, tv_ref, ti_ref):
    v = w_ref[...]
    r = lax.broadcasted_iota(jnp.int32, (B * ROWS, LANES), 0)
    c = lax.broadcasted_iota(jnp.int32, (B * ROWS, LANES), 1)
    lin = (r % ROWS) * LANES + c
    idx = lin

    size = 2
    while size <= N:
        d = size // 2
        while d >= 1:
            if d < LANES:
                ax, s = 1, d
            else:
                ax, s = 0, d // LANES
            low = (lin & d) == 0
            pv = jnp.where(low, jnp.roll(v, -s, axis=ax), jnp.roll(v, s, axis=ax))
            pi = jnp.where(low, jnp.roll(idx, -s, axis=ax), jnp.roll(idx, s, axis=ax))
            first = (v > pv) | ((v == pv) & (idx < pi))
            asc = (lin & size) != 0
            take_self = first ^ (~low) ^ asc
            v = jnp.where(take_self, v, pv)
            idx = jnp.where(take_self, idx, pi)
            d //= 2
        size *= 2

    blocks = []
    for b in range(B):
        blocks.append(idx[b * ROWS:b * ROWS + TOPROWS, :] + b * N)
        at_kth = lin[:ROWS, :] == (K - 1)
        vb = v[b * ROWS:(b + 1) * ROWS, :]
        ib = idx[b * ROWS:(b + 1) * ROWS, :]
        tv_ref[b] = jnp.sum(jnp.where(at_kth, vb, jnp.zeros_like(vb)))
        ti_ref[b] = jnp.sum(jnp.where(at_kth, ib, jnp.zeros_like(ib)))
    gidx_ref[...] = jnp.concatenate(blocks, axis=0)


def _topk_call(w2):
    return pl.pallas_call(
        _topk_body,
        out_shape=[
            jax.ShapeDtypeStruct((B * TOPROWS, LANES), jnp.int32),
            jax.ShapeDtypeStruct((B,), jnp.float32),
            jax.ShapeDtypeStruct((B,), jnp.int32),
        ],
        in_specs=[pl.BlockSpec((B * ROWS, LANES), lambda: (0, 0))],
        out_specs=[
            pl.BlockSpec((B * TOPROWS, LANES), lambda: (0, 0)),
            pl.BlockSpec(memory_space=pltpu.SMEM),
            pl.BlockSpec(memory_space=pltpu.SMEM),
        ],
    )(w2)


def _msum_body(tv_ref, ti_ref, a_ref, seq_ref, o_ref):
    b = pl.program_id(0)
    s = pl.program_id(1)
    t = tv_ref[b]
    bt = ti_ref[b]
    a = a_ref[pl.ds(b, 1), pl.ds(s * MR, MR)]
    j = lax.broadcasted_iota(jnp.int32, (1, MR), 1) + s * MR
    sel = (a > t) | ((a == t) & (j <= bt))
    m = jnp.where(sel, 0.0, 1.0).astype(jnp.bfloat16)
    part = jnp.dot(m, seq_ref[0].astype(jnp.bfloat16),
                   preferred_element_type=jnp.float32)

    @pl.when(s == 0)
    def _():
        o_ref[...] = jnp.zeros_like(o_ref)

    o_ref[...] += part[None]


def _masked_sum_call(tv, ti, attn, seq):
    return pl.pallas_call(
        _msum_body,
        out_shape=jax.ShapeDtypeStruct((B, 1, D), jnp.float32),
        grid=(B, N // MR),
        in_specs=[
            pl.BlockSpec(memory_space=pltpu.SMEM),
            pl.BlockSpec(memory_space=pltpu.SMEM),
            pl.BlockSpec((B, N), lambda b, s: (0, 0)),
            pl.BlockSpec((1, MR, D), lambda b, s: (b, s, 0)),
        ],
        out_specs=pl.BlockSpec((1, 1, D), lambda b, s: (b, 0, 0)),
        compiler_params=pltpu.CompilerParams(
            dimension_semantics=("parallel", "arbitrary")),
    )(tv, ti, attn, seq)


def _sc_gather_call(seq_flat, gidx2):
    mesh = plsc.VectorSubcoreMesh(core_axis_name="c", subcore_axis_name="s")
    L = 16

    @functools.partial(
        pl.kernel,
        out_type=jax.ShapeDtypeStruct(((K + 1) * B * SUB, LANES), jnp.float32),
        mesh=mesh,
        scratch_types=[
            pltpu.VMEM((B, PPW), jnp.int32),
            pltpu.VMEM((NSTEPS * SUB, CR), jnp.int32),
            pltpu.VMEM((CR, D), jnp.float32),
            pltpu.VMEM((CR, D), jnp.float32),
            pltpu.VMEM((CR, D), jnp.float32),
            pltpu.SemaphoreType.DMA,
            pltpu.SemaphoreType.DMA,
            pltpu.SemaphoreType.DMA,
            pltpu.SemaphoreType.DMA,
            pltpu.SemaphoreType.DMA,
            pltpu.SemaphoreType.DMA,
        ],
    )
    def k(seq_hbm, gidx_hbm, out_hbm,
          idx_v, dq_v, buf0, buf1, buf2, g0, g1, g2, w0, w1, w2):
        wid = lax.axis_index("s") * 2 + lax.axis_index("c")

        for b in range(B):
            pltpu.sync_copy(gidx_hbm.at[b * NW + wid], idx_v.at[b])

        iota = lax.broadcasted_iota(jnp.int32, (L,), 0) * (B * SUB)
        for c in range(NSTEPS):
            b, rb = c // (NSTEPS // B), c % (NSTEPS // B)
            base = (wid * PPW + rb * CR) * (B * SUB) + b
            for j in range(SUB):
                for h in range(CR // L):
                    dq_v[c * SUB + j, pl.ds(h * L, L)] = (
                        iota + (base + h * L * B * SUB + j * B))

        bufs = (buf0, buf1, buf2)
        gsems = (g0, g1, g2)
        wsems = (w0, w1, w2)

        def gather(t):
            b, rb = t // (NSTEPS // B), t % (NSTEPS // B)
            pltpu.make_async_copy(
                seq_hbm.at[idx_v.at[b, pl.ds(rb * CR, CR)]],
                bufs[t % NBUF], gsems[t % NBUF]).start()

        def gather_wait(t):
            b, rb = t // (NSTEPS // B), t % (NSTEPS // B)
            pltpu.make_async_copy(
                seq_hbm.at[idx_v.at[b, pl.ds(rb * CR, CR)]],
                bufs[t % NBUF], gsems[t % NBUF]).wait()

        def scatter(t, wait):
            buf, ws = bufs[t % NBUF], wsems[t % NBUF]
            for j in range(SUB):
                cp = pltpu.make_async_copy(
                    buf.at[:, pl.ds(j * LANES, LANES)],
                    out_hbm.at[dq_v.at[t * SUB + j]], ws)
                if wait:
                    cp.wait()
                else:
                    cp.start()

        gather(0)
        gather(1)
        for t in range(NSTEPS):
            gather_wait(t)
            scatter(t, wait=False)
            if t + 2 < NSTEPS:
                if t - 1 >= 0:
                    scatter(t - 1, wait=True)
                gather(t + 2)
        for t in range(NSTEPS - 3, NSTEPS):
            scatter(t, wait=True)

    return k(seq_flat, gidx2)


def _fin_body(orow_ref, rsum_ref, o_ref):
    cnt = jnp.float32(K) + jnp.float32(1e-10)
    row = jnp.float32(ALPHA) * (rsum_ref[...] / cnt)
    parts = []
    for q in range(B * SUB):
        j, b = q // B, q % B
        parts.append(row[b:b + 1, j * LANES:(j + 1) * LANES])
    o_ref[...] = jnp.concatenate(parts, axis=0)


def _finalize_call(out1, rsum):
    return pl.pallas_call(
        _fin_body,
        out_shape=jax.ShapeDtypeStruct(((K + 1) * B * SUB, LANES), jnp.float32),
        grid=(1,),
        in_specs=[
            pl.BlockSpec((B * SUB, LANES), lambda i: (K, 0)),
            pl.BlockSpec((B, D), lambda i: (0, 0)),
        ],
        out_specs=pl.BlockSpec((B * SUB, LANES), lambda i: (K, 0)),
        input_output_aliases={0: 0},
    )(out1, rsum)


def kernel(seq, attn_weights):
    if attn_weights.ndim == 3:
        attn_weights = jnp.squeeze(attn_weights, axis=1)
    w2 = attn_weights.reshape(B * ROWS, LANES)
    gidx, tv, ti = _topk_call(w2)
    rsum = _masked_sum_call(tv, ti, attn_weights, seq)
    out1 = _sc_gather_call(seq.reshape(B * N, D), gidx)
    out_flat = _finalize_call(out1, rsum.reshape(B, D))
    return (out_flat.reshape(K + 1, SUB, B, LANES)
            .transpose(2, 0, 1, 3)
            .reshape(B, K + 1, D))
